# Initial kernel scaffold; baseline (speedup 1.0000x reference)
import jax
import jax.numpy as jnp
from jax.experimental import pallas as pl


def kernel(loc_data, conf_data, occluded_data, targets, priors):
    raise NotImplementedError("write your pallas kernel here")


# speedup vs baseline: 37.0905x; 37.0905x over previous
import functools

import jax
import jax.numpy as jnp
from jax import lax
from jax.experimental import pallas as pl


_S = 8


def _mbl_kernel(loc_ref, conf_ref, occ_ref, pri_ref, tgt_ref, out_ref,
                *, P, L, C, NO):
    f32 = jnp.float32
    s_iota = lax.broadcasted_iota(jnp.int32, (_S, L), 0)
    l_iota = lax.broadcasted_iota(jnp.int32, (_S, L), 1)
    gidx = s_iota * L + l_iota
    valid = gidx < P

    cx = pri_ref[0]
    cy = pri_ref[1]
    w = pri_ref[2]
    h = pri_ref[3]
    px0 = cx - w * 0.5
    py0 = cy - h * 0.5
    px1 = cx + w * 0.5
    py1 = cy + h * 0.5
    parea = (px1 - px0) * (py1 - py0)

    tg = tgt_ref[0]

    bto = jnp.full((_S, L), -2.0, f32)
    bti = jnp.zeros((_S, L), jnp.int32)
    bpi = []
    for j in range(NO):
        tx0 = tg[0, j]
        ty0 = tg[1, j]
        tx1 = tg[2, j]
        ty1 = tg[3, j]
        iw = jnp.maximum(jnp.minimum(px1, tx1) - jnp.maximum(px0, tx0), 0.0)
        ih = jnp.maximum(jnp.minimum(py1, ty1) - jnp.maximum(py0, ty0), 0.0)
        inter = iw * ih
        ta = (tx1 - tx0) * (ty1 - ty0)
        ov = inter / (ta + parea - inter)
        ov = jnp.where(valid, ov, -1.0)
        m = jnp.max(ov)
        bpi.append(jnp.min(jnp.where(ov == m, gidx, P)))
        upd = ov > bto
        bto = jnp.where(upd, ov, bto)
        bti = jnp.where(upd, j, bti)

    for j in range(NO):
        msk = gidx == bpi[j]
        bto = jnp.where(msk, 2.0, bto)
        bti = jnp.where(msk, j, bti)

    acc = [jnp.full((_S, L), tg[r, 0], f32) for r in range(6)]
    for j in range(1, NO):
        msk = bti == j
        for r in range(6):
            acc[r] = jnp.where(msk, tg[r, j], acc[r])
    mx0, my0, mx1, my1, lab, tiou = acc

    conf_t = jnp.where(bto < 0.5, 0, lab.astype(jnp.int32) + 1)
    pos = conf_t > 0
    npos = jnp.sum(jnp.where(pos, 1.0, 0.0))

    g_cx = ((mx0 + mx1) * 0.5 - cx) / (0.1 * w)
    g_cy = ((my0 + my1) * 0.5 - cy) / (0.1 * h)
    g_w = jnp.log((mx1 - mx0) / w) / 0.2
    g_h = jnp.log((my1 - my0) / h) / 0.2
    sl1 = jnp.zeros((_S, L), f32)
    for d, g in enumerate((g_cx, g_cy, g_w, g_h)):
        diff = loc_ref[0, d] - g
        ad = jnp.abs(diff)
        sl1 = sl1 + jnp.where(ad < 1.0, 0.5 * diff * diff, ad - 0.5)
    loss_l = jnp.sum(jnp.where(pos, sl1, 0.0))

    occv = occ_ref[0, 0]
    iou_p = jnp.where(tiou == -1.0, -1.0, occv)
    loss_o = jnp.sum(jnp.where(pos, (iou_p - tiou) ** 2, 0.0))

    cf = conf_ref[0]
    mxv = jnp.max(cf, axis=0)
    sm = jnp.sum(jnp.exp(cf - mxv[None]), axis=0)
    lse = jnp.log(sm) + mxv
    cls_iota = lax.broadcasted_iota(jnp.int32, (C, _S, L), 0)
    gathered = jnp.sum(jnp.where(cls_iota == conf_t[None], cf, 0.0), axis=0)
    ce = lse - gathered
    sum_pos_ce = jnp.sum(jnp.where(pos, ce, 0.0))
    mine = jnp.where(pos | jnp.logical_not(valid), 0.0, ce)

    k_f = jnp.minimum(npos * 3.0, jnp.float32(P - 1))

    def body(i, cand):
        cand2 = cand | (jnp.int32(1) << (jnp.int32(30) - i))
        thr = lax.bitcast_convert_type(
            jnp.broadcast_to(cand2, (_S, L)), f32)
        cnt = jnp.sum(jnp.where(mine >= thr, 1.0, 0.0))
        return jnp.where(cnt >= k_f, cand2, cand)

    cand = lax.fori_loop(0, 31, body, jnp.int32(0))
    thr_v = lax.bitcast_convert_type(jnp.broadcast_to(cand, (_S, L)), f32)
    thr = jnp.max(thr_v)
    cnt_gt = jnp.sum(jnp.where(mine > thr_v, 1.0, 0.0))
    sum_gt = jnp.sum(jnp.where(mine > thr_v, mine, 0.0))
    loss_c = sum_pos_ce + sum_gt + (k_f - cnt_gt) * thr

    o_iota = lax.broadcasted_iota(jnp.int32, (1, 128), 1)
    outv = jnp.where(o_iota == 0, loss_l, 0.0)
    outv = jnp.where(o_iota == 1, loss_c, outv)
    outv = jnp.where(o_iota == 2, loss_o, outv)
    outv = jnp.where(o_iota == 3, npos, outv)
    out_ref[0] = outv


def kernel(loc_data, conf_data, occluded_data, targets, priors):
    B, P, C = conf_data.shape
    NO = targets.shape[1]
    p_pad = ((P + _S * 128 - 1) // (_S * 128)) * (_S * 128)
    L = p_pad // _S
    pad = p_pad - P

    locT = jnp.pad(jnp.transpose(loc_data, (0, 2, 1)),
                   ((0, 0), (0, 0), (0, pad))).reshape(B, 4, _S, L)
    confT = jnp.pad(jnp.transpose(conf_data, (0, 2, 1)),
                    ((0, 0), (0, 0), (0, pad))).reshape(B, C, _S, L)
    occT = jnp.pad(occluded_data.reshape(B, 1, P),
                   ((0, 0), (0, 0), (0, pad))).reshape(B, 1, _S, L)
    priT = jnp.pad(priors.T, ((0, 0), (0, pad))).reshape(4, _S, L)
    tgtT = jnp.transpose(targets, (0, 2, 1))

    out = pl.pallas_call(
        functools.partial(_mbl_kernel, P=P, L=L, C=C, NO=NO),
        grid=(B,),
        in_specs=[
            pl.BlockSpec((1, 4, _S, L), lambda b: (b, 0, 0, 0)),
            pl.BlockSpec((1, C, _S, L), lambda b: (b, 0, 0, 0)),
            pl.BlockSpec((1, 1, _S, L), lambda b: (b, 0, 0, 0)),
            pl.BlockSpec((4, _S, L), lambda b: (0, 0, 0)),
            pl.BlockSpec((1, 6, NO), lambda b: (b, 0, 0)),
        ],
        out_specs=pl.BlockSpec((1, 1, 128), lambda b: (b, 0, 0)),
        out_shape=jax.ShapeDtypeStruct((B, 1, 128), jnp.float32),
    )(locT, confT, occT, priT, tgtT)

    loss_l = jnp.sum(out[:, 0, 0])
    loss_c = jnp.sum(out[:, 0, 1])
    loss_o = jnp.sum(out[:, 0, 2])
    n = jnp.sum(out[:, 0, 3])
    return (loss_l / n, loss_c / n, loss_o / n)


# speedup vs baseline: 37.3086x; 1.0059x over previous
import functools

import jax
import jax.numpy as jnp
from jax import lax
from jax.experimental import pallas as pl


_S = 8


def _mbl_kernel(loc_ref, conf_ref, occ_ref, pri_ref, tgt_ref, out_ref,
                *, P, L, C, NO):
    f32 = jnp.float32
    s_iota = lax.broadcasted_iota(jnp.int32, (_S, L), 0)
    l_iota = lax.broadcasted_iota(jnp.int32, (_S, L), 1)
    gidx = s_iota * L + l_iota
    valid = gidx < P

    cx = pri_ref[0]
    cy = pri_ref[1]
    w = pri_ref[2]
    h = pri_ref[3]
    px0 = cx - w * 0.5
    py0 = cy - h * 0.5
    px1 = cx + w * 0.5
    py1 = cy + h * 0.5
    parea = (px1 - px0) * (py1 - py0)

    tg = tgt_ref[0]

    bto = jnp.full((_S, L), -2.0, f32)
    bti = jnp.zeros((_S, L), jnp.int32)
    bpi = []
    for j in range(NO):
        tx0 = tg[0, j]
        ty0 = tg[1, j]
        tx1 = tg[2, j]
        ty1 = tg[3, j]
        iw = jnp.maximum(jnp.minimum(px1, tx1) - jnp.maximum(px0, tx0), 0.0)
        ih = jnp.maximum(jnp.minimum(py1, ty1) - jnp.maximum(py0, ty0), 0.0)
        inter = iw * ih
        ta = (tx1 - tx0) * (ty1 - ty0)
        ov = inter / (ta + parea - inter)
        ov = jnp.where(valid, ov, -1.0)
        m = jnp.max(ov)
        bpi.append(jnp.min(jnp.where(ov == m, gidx, P)))
        upd = ov > bto
        bto = jnp.where(upd, ov, bto)
        bti = jnp.where(upd, j, bti)

    for j in range(NO):
        msk = gidx == bpi[j]
        bto = jnp.where(msk, 2.0, bto)
        bti = jnp.where(msk, j, bti)

    acc = [jnp.full((_S, L), tg[r, 0], f32) for r in range(6)]
    for j in range(1, NO):
        msk = bti == j
        for r in range(6):
            acc[r] = jnp.where(msk, tg[r, j], acc[r])
    mx0, my0, mx1, my1, lab, tiou = acc

    conf_t = jnp.where(bto < 0.5, 0, lab.astype(jnp.int32) + 1)
    pos = conf_t > 0
    npos = jnp.sum(jnp.where(pos, 1.0, 0.0))

    g_cx = ((mx0 + mx1) * 0.5 - cx) / (0.1 * w)
    g_cy = ((my0 + my1) * 0.5 - cy) / (0.1 * h)
    g_w = jnp.log((mx1 - mx0) / w) / 0.2
    g_h = jnp.log((my1 - my0) / h) / 0.2
    sl1 = jnp.zeros((_S, L), f32)
    for d, g in enumerate((g_cx, g_cy, g_w, g_h)):
        diff = loc_ref[0, d] - g
        ad = jnp.abs(diff)
        sl1 = sl1 + jnp.where(ad < 1.0, 0.5 * diff * diff, ad - 0.5)
    loss_l = jnp.sum(jnp.where(pos, sl1, 0.0))

    occv = occ_ref[0, 0]
    iou_p = jnp.where(tiou == -1.0, -1.0, occv)
    loss_o = jnp.sum(jnp.where(pos, (iou_p - tiou) ** 2, 0.0))

    cf = conf_ref[0]
    sm = jnp.sum(jnp.exp(cf), axis=0)
    lse = jnp.log(sm)
    planes = [cf[c] for c in range(C)]
    idx = conf_t
    while len(planes) > 1:
        bit0 = (idx & 1) == 1
        nxt = []
        for c in range(0, len(planes) - 1, 2):
            nxt.append(jnp.where(bit0, planes[c + 1], planes[c]))
        if len(planes) % 2 == 1:
            nxt.append(planes[-1])
        planes = nxt
        idx = idx >> 1
    gathered = planes[0]
    ce = lse - gathered
    sum_pos_ce = jnp.sum(jnp.where(pos, ce, 0.0))
    mine = jnp.where(pos | jnp.logical_not(valid), 0.0, ce)

    k_f = jnp.minimum(npos * 3.0, jnp.float32(P - 1))

    def body(i, cand):
        cand2 = cand | (jnp.int32(1) << (jnp.int32(30) - i))
        thr = lax.bitcast_convert_type(
            jnp.broadcast_to(cand2, (_S, L)), f32)
        cnt = jnp.sum(jnp.where(mine >= thr, 1.0, 0.0))
        return jnp.where(cnt >= k_f, cand2, cand)

    cand = lax.fori_loop(0, 31, body, jnp.int32(0))
    thr_v = lax.bitcast_convert_type(jnp.broadcast_to(cand, (_S, L)), f32)
    thr = jnp.max(thr_v)
    cnt_gt = jnp.sum(jnp.where(mine > thr_v, 1.0, 0.0))
    sum_gt = jnp.sum(jnp.where(mine > thr_v, mine, 0.0))
    loss_c = sum_pos_ce + sum_gt + (k_f - cnt_gt) * thr

    o_iota = lax.broadcasted_iota(jnp.int32, (1, 128), 1)
    outv = jnp.where(o_iota == 0, loss_l, 0.0)
    outv = jnp.where(o_iota == 1, loss_c, outv)
    outv = jnp.where(o_iota == 2, loss_o, outv)
    outv = jnp.where(o_iota == 3, npos, outv)
    out_ref[0] = outv


def kernel(loc_data, conf_data, occluded_data, targets, priors):
    B, P, C = conf_data.shape
    NO = targets.shape[1]
    p_pad = ((P + _S * 128 - 1) // (_S * 128)) * (_S * 128)
    L = p_pad // _S
    pad = p_pad - P

    locT = jnp.pad(jnp.transpose(loc_data, (0, 2, 1)),
                   ((0, 0), (0, 0), (0, pad))).reshape(B, 4, _S, L)
    confT = jnp.pad(jnp.transpose(conf_data, (0, 2, 1)),
                    ((0, 0), (0, 0), (0, pad))).reshape(B, C, _S, L)
    occT = jnp.pad(occluded_data.reshape(B, 1, P),
                   ((0, 0), (0, 0), (0, pad))).reshape(B, 1, _S, L)
    priT = jnp.pad(priors.T, ((0, 0), (0, pad))).reshape(4, _S, L)
    tgtT = jnp.transpose(targets, (0, 2, 1))

    out = pl.pallas_call(
        functools.partial(_mbl_kernel, P=P, L=L, C=C, NO=NO),
        grid=(B,),
        in_specs=[
            pl.BlockSpec((1, 4, _S, L), lambda b: (b, 0, 0, 0)),
            pl.BlockSpec((1, C, _S, L), lambda b: (b, 0, 0, 0)),
            pl.BlockSpec((1, 1, _S, L), lambda b: (b, 0, 0, 0)),
            pl.BlockSpec((4, _S, L), lambda b: (0, 0, 0)),
            pl.BlockSpec((1, 6, NO), lambda b: (b, 0, 0)),
        ],
        out_specs=pl.BlockSpec((1, 1, 128), lambda b: (b, 0, 0)),
        out_shape=jax.ShapeDtypeStruct((B, 1, 128), jnp.float32),
    )(locT, confT, occT, priT, tgtT)

    loss_l = jnp.sum(out[:, 0, 0])
    loss_c = jnp.sum(out[:, 0, 1])
    loss_o = jnp.sum(out[:, 0, 2])
    n = jnp.sum(out[:, 0, 3])
    return (loss_l / n, loss_c / n, loss_o / n)
